# baseline (speedup 1.0000x reference)
import functools

import jax
import jax.numpy as jnp
from jax import lax
from jax.experimental import pallas as pl
from jax.experimental.pallas import tpu as pltpu
from jax.experimental.pallas import tpu_sc as plsc

_B = 16
_MAX_LEN = 4096
_TOTAL = 32768
_HALF = _MAX_LEN // 2
_BUF = _HALF + 16
_LANES = 16


def _pad_sc_body(flat_hbm, cu_hbm, out_hbm, cu_v, buf_v, out_v):
    h = lax.axis_index("c")
    b = lax.axis_index("s")

    pltpu.sync_copy(cu_hbm.at[pl.ds(0, _LANES)], cu_v)
    lane = lax.iota(jnp.int32, _LANES)
    b_v = jnp.full((_LANES,), b, dtype=jnp.int32)
    cu_pair = plsc.load_gather(
        cu_v, [jnp.minimum(b_v + jnp.minimum(lane, 1), _LANES - 1)]
    )
    row_start = cu_pair[0]
    row_end = jnp.where(b == _B - 1, _TOTAL, cu_pair[1])

    start = row_start + h * _HALF
    n_valid = jnp.clip(row_end - start, 0, _HALF)

    aligned = (start // 8) * 8
    read_start = jnp.minimum(aligned, _TOTAL - _BUF)
    read_start = pl.multiple_of(read_start, 8)
    off = start - read_start

    pltpu.sync_copy(flat_hbm.at[pl.ds(read_start, _BUF)], buf_v)

    off_v = jnp.full((_LANES,), off, dtype=jnp.int32)
    nv_v = jnp.full((_LANES,), n_valid, dtype=jnp.int32)

    def body(j, _):
        pos = j * _LANES + lane
        mask = pos < nv_v
        idx = jnp.minimum(pos + off_v, _BUF - 1)
        vals = plsc.load_gather(buf_v, [idx])
        vals = jnp.where(mask, vals, 0.0)
        out_v[pl.ds(j * _LANES, _LANES)] = vals
        return 0

    lax.fori_loop(0, _HALF // _LANES, body, 0, unroll=8)

    dst = b * _MAX_LEN + h * _HALF
    dst = pl.multiple_of(dst, _HALF)
    pltpu.sync_copy(out_v, out_hbm.at[pl.ds(dst, _HALF)])


@functools.cache
def _build_kernel():
    mesh = plsc.VectorSubcoreMesh(core_axis_name="c", subcore_axis_name="s")
    return pl.kernel(
        _pad_sc_body,
        out_type=jax.ShapeDtypeStruct((_B * _MAX_LEN,), jnp.float32),
        mesh=mesh,
        scratch_types=[
            pltpu.VMEM((_LANES,), jnp.int32),
            pltpu.VMEM((_BUF,), jnp.float32),
            pltpu.VMEM((_HALF,), jnp.float32),
        ],
        compiler_params=pltpu.CompilerParams(
            needs_layout_passes=False,
            disable_bounds_checks=True,
            disable_semaphore_checks=True,
            skip_device_barrier=True,
        ),
    )


def kernel(flat, cu_seqlens):
    out = _build_kernel()(flat, cu_seqlens.astype(jnp.int32))
    return out.reshape(_B, _MAX_LEN)
